# Initial kernel scaffold; baseline (speedup 1.0000x reference)
import jax
import jax.numpy as jnp
from jax.experimental import pallas as pl


def kernel(xyz0, xyz1, xyz2, x, W1, b1, W2, b2, Wf1, bf1, Wf2, bf2, Wf3, bf3):
    raise NotImplementedError("write your pallas kernel here")


# speedup vs baseline: 26.1887x; 26.1887x over previous
import jax
import jax.numpy as jnp
from jax import lax
from jax.experimental import pallas as pl
from jax.experimental.pallas import tpu as pltpu


def _top3_weight_matrix(d):
    n, s = d.shape
    iota = lax.broadcasted_iota(jnp.int32, (n, s), 1)
    big = jnp.float32(1e9)
    sbig = jnp.int32(2 * s)
    dwork = d
    sels = []
    recips = []
    for k in range(3):
        m = jnp.min(dwork, axis=-1, keepdims=True)
        elig = dwork == m
        idxm = jnp.min(jnp.where(elig, iota, sbig), axis=-1, keepdims=True)
        sel = iota == idxm
        sels.append(sel)
        recips.append(1.0 / (jnp.maximum(m, 0.0) + 1e-8))
        if k < 2:
            dwork = jnp.where(sel, big, dwork)
    rnorm = 1.0 / (recips[0] + recips[1] + recips[2])
    w = jnp.zeros_like(d)
    for k in range(3):
        w = jnp.where(sels[k], recips[k], w)
    return w * rnorm


def _sqdist(q, rt):
    qq = jnp.sum(q * q, axis=-1, keepdims=True)
    rr = jnp.sum(rt * rt, axis=0, keepdims=True)
    qr = jnp.dot(q, rt, preferred_element_type=jnp.float32)
    return qq + rr - 2.0 * qr


def _stage1_body(xyz1_ref, x2t_ref, feat_ref, w1_ref, b1_ref, h1_ref):
    d = _sqdist(xyz1_ref[0], x2t_ref[0])
    w = _top3_weight_matrix(d)
    interp = jnp.dot(w, feat_ref[0], preferred_element_type=jnp.float32)
    h = jnp.dot(interp, w1_ref[...], preferred_element_type=jnp.float32)
    h1_ref[0] = jnp.maximum(h + b1_ref[...], 0.0)


def _stage2_body(xyz0_ref, x1t_ref, h1_ref, w2_ref, b2_ref,
                 wf1_ref, bf1_ref, wf2_ref, bf2_ref, wf3_ref, bf3_ref,
                 x1_ref, x2_ref):
    d = _sqdist(xyz0_ref[0], x1t_ref[0])
    w = _top3_weight_matrix(d)
    interp = jnp.dot(w, h1_ref[0], preferred_element_type=jnp.float32)
    h = jnp.maximum(
        jnp.dot(interp, w2_ref[...], preferred_element_type=jnp.float32)
        + b2_ref[...], 0.0)
    hf = jnp.maximum(
        jnp.dot(h, wf1_ref[...], preferred_element_type=jnp.float32)
        + bf1_ref[...], 0.0)
    x1_ref[0] = jnp.maximum(
        jnp.dot(hf, wf2_ref[...], preferred_element_type=jnp.float32)
        + bf2_ref[...], 0.0)
    x2_ref[0] = jnp.maximum(
        jnp.dot(hf, wf3_ref[...], preferred_element_type=jnp.float32)
        + bf3_ref[...], 0.0)


def _full(shape):
    return pl.BlockSpec(shape, lambda *_: tuple(0 for _ in shape))


def kernel(xyz0, xyz1, xyz2, x, W1, b1, W2, b2, Wf1, bf1, Wf2, bf2, Wf3, bf3):
    B, N0, _ = xyz0.shape
    N1 = xyz1.shape[1]
    N2 = xyz2.shape[1]
    BLK = 512

    x1t = jnp.swapaxes(xyz1, 1, 2)
    x2t = jnp.swapaxes(xyz2, 1, 2)
    b1r = b1.reshape(1, -1)
    b2r = b2.reshape(1, -1)
    bf1r = bf1.reshape(1, -1)
    bf2r = bf2.reshape(1, -1)
    bf3r = bf3.reshape(1, -1)

    h1 = pl.pallas_call(
        _stage1_body,
        grid=(B,),
        in_specs=[
            pl.BlockSpec((1, N1, 3), lambda b: (b, 0, 0)),
            pl.BlockSpec((1, 3, N2), lambda b: (b, 0, 0)),
            pl.BlockSpec((1, N2, 6), lambda b: (b, 0, 0)),
            pl.BlockSpec(W1.shape, lambda b: (0, 0)),
            pl.BlockSpec((1, 12), lambda b: (0, 0)),
        ],
        out_specs=pl.BlockSpec((1, N1, 12), lambda b: (b, 0, 0)),
        out_shape=jax.ShapeDtypeStruct((B, N1, 12), jnp.float32),
    )(xyz1, x2t, x, W1, b1r)

    grid2 = (B, N0 // BLK)
    x1o, x2o = pl.pallas_call(
        _stage2_body,
        grid=grid2,
        in_specs=[
            pl.BlockSpec((1, BLK, 3), lambda b, j: (b, j, 0)),
            pl.BlockSpec((1, 3, N1), lambda b, j: (b, 0, 0)),
            pl.BlockSpec((1, N1, 12), lambda b, j: (b, 0, 0)),
            pl.BlockSpec(W2.shape, lambda b, j: (0, 0)),
            pl.BlockSpec((1, 12), lambda b, j: (0, 0)),
            pl.BlockSpec(Wf1.shape, lambda b, j: (0, 0)),
            pl.BlockSpec((1, 24), lambda b, j: (0, 0)),
            pl.BlockSpec(Wf2.shape, lambda b, j: (0, 0)),
            pl.BlockSpec((1, 8), lambda b, j: (0, 0)),
            pl.BlockSpec(Wf3.shape, lambda b, j: (0, 0)),
            pl.BlockSpec((1, 8), lambda b, j: (0, 0)),
        ],
        out_specs=[
            pl.BlockSpec((1, BLK, 8), lambda b, j: (b, j, 0)),
            pl.BlockSpec((1, BLK, 8), lambda b, j: (b, j, 0)),
        ],
        out_shape=[
            jax.ShapeDtypeStruct((B, N0, 8), jnp.float32),
            jax.ShapeDtypeStruct((B, N0, 8), jnp.float32),
        ],
    )(xyz0, x1t, h1, W2, b2r, Wf1, bf1r, Wf2, bf2r, Wf3, bf3r)

    return (x1o, x2o)


# speedup vs baseline: 37.2360x; 1.4218x over previous
import jax
import jax.numpy as jnp
from jax import lax
from jax.experimental import pallas as pl
from jax.experimental.pallas import tpu as pltpu

_IDX_BITS = 10
_IDX_MASK = (1 << _IDX_BITS) - 1


def _top3_weight_matrix_t(d):
    s, n = d.shape
    iota = lax.broadcasted_iota(jnp.int32, (s, n), 0)
    dc = jnp.maximum(d, 0.0)
    key0 = (lax.bitcast_convert_type(dc, jnp.int32) & jnp.int32(~_IDX_MASK)) | iota
    big = jnp.int32(0x7FFFFFFF)

    k1 = jnp.min(key0, axis=0, keepdims=True)
    key1 = jnp.where(key0 == k1, big, key0)
    k2 = jnp.min(key1, axis=0, keepdims=True)
    key2 = jnp.where(key1 == k2, big, key1)
    k3 = jnp.min(key2, axis=0, keepdims=True)

    def recip(k):
        dist = lax.bitcast_convert_type(k & jnp.int32(~_IDX_MASK), jnp.float32)
        return 1.0 / (dist + 1e-8)

    r1, r2, r3 = recip(k1), recip(k2), recip(k3)
    rnorm = 1.0 / (r1 + r2 + r3)
    w = jnp.where(key0 == k1, r1, 0.0)
    w = jnp.where(key0 == k2, r2, w)
    w = jnp.where(key0 == k3, r3, w)
    return w * rnorm


def _sqdist_t(r, qt):
    rr = jnp.sum(r * r, axis=-1, keepdims=True)
    qq = jnp.sum(qt * qt, axis=0, keepdims=True)
    rq = jnp.dot(r, qt, preferred_element_type=jnp.float32)
    return rr + qq - 2.0 * rq


def _stage1_body(x1t_ref, xyz2_ref, xt_ref, w1t_ref, b1t_ref, h1t_ref):
    d = _sqdist_t(xyz2_ref[0], x1t_ref[0])
    w = _top3_weight_matrix_t(d)
    interp = jnp.dot(xt_ref[0], w, preferred_element_type=jnp.float32)
    h = jnp.dot(w1t_ref[...], interp, preferred_element_type=jnp.float32)
    h1t_ref[0] = jnp.maximum(h + b1t_ref[...], 0.0)


def _stage2_body(x0t_ref, xyz1_ref, h1t_ref, w2t_ref, b2t_ref,
                 wf1t_ref, bf1t_ref, wf2t_ref, bf2t_ref, wf3t_ref, bf3t_ref,
                 x1_ref, x2_ref):
    d = _sqdist_t(xyz1_ref[0], x0t_ref[0])
    w = _top3_weight_matrix_t(d)
    interp = jnp.dot(h1t_ref[0], w, preferred_element_type=jnp.float32)
    h = jnp.maximum(
        jnp.dot(w2t_ref[...], interp, preferred_element_type=jnp.float32)
        + b2t_ref[...], 0.0)
    hf = jnp.maximum(
        jnp.dot(wf1t_ref[...], h, preferred_element_type=jnp.float32)
        + bf1t_ref[...], 0.0)
    x1_ref[0] = jnp.maximum(
        jnp.dot(wf2t_ref[...], hf, preferred_element_type=jnp.float32)
        + bf2t_ref[...], 0.0)
    x2_ref[0] = jnp.maximum(
        jnp.dot(wf3t_ref[...], hf, preferred_element_type=jnp.float32)
        + bf3t_ref[...], 0.0)


def kernel(xyz0, xyz1, xyz2, x, W1, b1, W2, b2, Wf1, bf1, Wf2, bf2, Wf3, bf3):
    B, N0, _ = xyz0.shape
    N1 = xyz1.shape[1]
    N2 = xyz2.shape[1]
    BLK = 512

    x0t = jnp.swapaxes(xyz0, 1, 2)
    x1t = jnp.swapaxes(xyz1, 1, 2)
    xt = jnp.swapaxes(x, 1, 2)
    w1t = W1.T
    w2t = W2.T
    wf1t = Wf1.T
    wf2t = Wf2.T
    wf3t = Wf3.T
    b1t = b1.reshape(-1, 1)
    b2t = b2.reshape(-1, 1)
    bf1t = bf1.reshape(-1, 1)
    bf2t = bf2.reshape(-1, 1)
    bf3t = bf3.reshape(-1, 1)

    h1t = pl.pallas_call(
        _stage1_body,
        grid=(B,),
        in_specs=[
            pl.BlockSpec((1, 3, N1), lambda b: (b, 0, 0)),
            pl.BlockSpec((1, N2, 3), lambda b: (b, 0, 0)),
            pl.BlockSpec((1, 6, N2), lambda b: (b, 0, 0)),
            pl.BlockSpec(w1t.shape, lambda b: (0, 0)),
            pl.BlockSpec((12, 1), lambda b: (0, 0)),
        ],
        out_specs=pl.BlockSpec((1, 12, N1), lambda b: (b, 0, 0)),
        out_shape=jax.ShapeDtypeStruct((B, 12, N1), jnp.float32),
    )(x1t, xyz2, xt, w1t, b1t)

    grid2 = (B, N0 // BLK)
    x1o, x2o = pl.pallas_call(
        _stage2_body,
        grid=grid2,
        in_specs=[
            pl.BlockSpec((1, 3, BLK), lambda b, j: (b, 0, j)),
            pl.BlockSpec((1, N1, 3), lambda b, j: (b, 0, 0)),
            pl.BlockSpec((1, 12, N1), lambda b, j: (b, 0, 0)),
            pl.BlockSpec(w2t.shape, lambda b, j: (0, 0)),
            pl.BlockSpec((12, 1), lambda b, j: (0, 0)),
            pl.BlockSpec(wf1t.shape, lambda b, j: (0, 0)),
            pl.BlockSpec((24, 1), lambda b, j: (0, 0)),
            pl.BlockSpec(wf2t.shape, lambda b, j: (0, 0)),
            pl.BlockSpec((8, 1), lambda b, j: (0, 0)),
            pl.BlockSpec(wf3t.shape, lambda b, j: (0, 0)),
            pl.BlockSpec((8, 1), lambda b, j: (0, 0)),
        ],
        out_specs=[
            pl.BlockSpec((1, 8, BLK), lambda b, j: (b, 0, j)),
            pl.BlockSpec((1, 8, BLK), lambda b, j: (b, 0, j)),
        ],
        out_shape=[
            jax.ShapeDtypeStruct((B, 8, N0), jnp.float32),
            jax.ShapeDtypeStruct((B, 8, N0), jnp.float32),
        ],
    )(x0t, xyz1, h1t, w2t, b2t, wf1t, bf1t, wf2t, bf2t, wf3t, bf3t)

    return (jnp.swapaxes(x1o, 1, 2), jnp.swapaxes(x2o, 1, 2))
